# baseline (speedup 1.0000x reference)
import functools
import math

import jax
import jax.numpy as jnp
from jax import lax
from jax.experimental import pallas as pl
from jax.experimental.pallas import tpu as pltpu
from jax.experimental.pallas import tpu_sc as plsc

N = 1024
V = 100000
PAD = 0
EPS = 0.1
CONF = 1.0 - EPS
U = EPS / (V - 2)
C0 = EPS * math.log(U) + CONF * math.log(CONF)


NW = 32
RPW = N // NW
L = 16


@functools.partial(
    pl.kernel,
    mesh=plsc.VectorSubcoreMesh(core_axis_name="c", subcore_axis_name="s"),
    out_type=jax.ShapeDtypeStruct((N,), jnp.float32),
    scratch_types=[
        pltpu.VMEM((RPW,), jnp.int32),
        pltpu.VMEM((2 * RPW,), jnp.int32),
        pltpu.VMEM((2 * RPW,), jnp.float32),
        pltpu.VMEM((RPW,), jnp.float32),
        pltpu.SemaphoreType.DMA,
    ],
)
def _sc_rowconst(pflat_hbm, trg_hbm, out_hbm, trg_v, idx_v, vals_v, r_v, sem):
    wid = lax.axis_index("s") * 2 + lax.axis_index("c")
    base = wid * RPW
    pltpu.sync_copy(trg_hbm.at[pl.ds(base, RPW)], trg_v)
    for k in range(RPW // L):
        t = trg_v[pl.ds(k * L, L)]
        rows = lax.iota(jnp.int32, L) + (base + k * L)
        idx_v[pl.ds(k * L, L)] = rows * V + t
        idx_v[pl.ds(RPW + k * L, L)] = rows * V
    pltpu.async_copy(pflat_hbm.at[idx_v], vals_v, sem).wait()
    for k in range(RPW // L):
        g = vals_v[pl.ds(k * L, L)]
        p0 = vals_v[pl.ds(RPW + k * L, L)]
        t = trg_v[pl.ds(k * L, L)]
        r = C0 + U * p0 + (U - CONF) * g
        r_v[pl.ds(k * L, L)] = jnp.where(t != PAD, r, 0.0)
    pltpu.sync_copy(r_v, out_hbm.at[pl.ds(base, RPW)])


CB = 2048
NBLK = (V + CB - 1) // CB
AW = 512
NFOLD = CB // AW


def _tc_body(trg_ref, r_ref, x_ref, out_ref, acc_ref):
    j = pl.program_id(0)
    x = x_ref[...]

    @pl.when(j == 0)
    def _init():
        acc = x[:, 0:AW]
        for f in range(1, NFOLD):
            acc = acc + x[:, f * AW:(f + 1) * AW]
        acc_ref[...] = acc

    @pl.when(jnp.logical_and(j > 0, j < NBLK - 1))
    def _accum():
        acc = x[:, 0:AW]
        for f in range(1, NFOLD):
            acc = acc + x[:, f * AW:(f + 1) * AW]
        acc_ref[...] += acc

    @pl.when(j == NBLK - 1)
    def _final():
        col = jax.lax.broadcasted_iota(jnp.int32, (N, CB), 1) + j * CB
        xv = jnp.where(col < V, x, 0.0)
        acc = xv[:, 0:AW]
        for f in range(1, NFOLD):
            acc = acc + xv[:, f * AW:(f + 1) * AW]
        acc = acc_ref[...] + acc
        s = jnp.sum(acc, axis=1, keepdims=True)
        s = jnp.where(trg_ref[...] != PAD, s, 0.0)
        out_ref[0, 0] = (jnp.sum(r_ref[...]) - U * jnp.sum(s)) / N


def kernel(preds, trgs):
    trgs32 = trgs.astype(jnp.int32)
    r = _sc_rowconst(preds.reshape(-1), trgs32)
    out = pl.pallas_call(
        _tc_body,
        grid=(NBLK,),
        in_specs=[
            pl.BlockSpec((N, 1), lambda j: (0, 0)),
            pl.BlockSpec((8, 128), lambda j: (0, 0)),
            pl.BlockSpec((N, CB), lambda j: (0, j)),
        ],
        out_specs=pl.BlockSpec((1, 1), lambda j: (0, 0), memory_space=pltpu.SMEM),
        out_shape=jax.ShapeDtypeStruct((1, 1), jnp.float32),
        scratch_shapes=[
            pltpu.VMEM((N, AW), jnp.float32),
        ],
        compiler_params=pltpu.CompilerParams(
            dimension_semantics=("arbitrary",),
        ),
    )(trgs32.reshape(N, 1), r.reshape(8, 128), preds)
    return out[0, 0]


# speedup vs baseline: 1.8282x; 1.8282x over previous
import math

import jax
import jax.numpy as jnp
from jax.experimental import pallas as pl
from jax.experimental.pallas import tpu as pltpu

N = 1024
V = 100000
PAD = 0
EPS = 0.1
CONF = 1.0 - EPS
U = EPS / (V - 2)
C0 = EPS * math.log(U) + CONF * math.log(CONF)

CB = 2048
NBLK = (V + CB - 1) // CB
F = CB // 128
NFULL = (V - (NBLK - 1) * CB) // 128
TAILW = V - (NBLK - 1) * CB - NFULL * 128


def _fold(x, td, j, sa_ref, ga_ref, nslice, init):
    sa = x[:, 0:128]
    ga = jnp.where(td == j * F, x[:, 0:128], 0.0)
    for f in range(1, nslice):
        xs = x[:, f * 128:(f + 1) * 128]
        sa = sa + xs
        ga = ga + jnp.where(td == j * F + f, xs, 0.0)
    if init:
        sa_ref[...] = sa
        ga_ref[...] = ga
    else:
        sa_ref[...] += sa
        ga_ref[...] += ga


def _tc_body(td_ref, tm_ref, x_ref, out_ref, sa_ref, ga_ref, p0_ref):
    j = pl.program_id(0)
    x = x_ref[...]
    td = td_ref[...]
    tm = tm_ref[...]

    @pl.when(j == 0)
    def _init():
        _fold(x, td, 0, sa_ref, ga_ref, F, True)
        p0_ref[...] = x[:, 0:1]

    @pl.when(jnp.logical_and(j > 0, j < NBLK - 1))
    def _accum():
        _fold(x, td, j, sa_ref, ga_ref, F, False)

    @pl.when(j == NBLK - 1)
    def _final():
        jj = NBLK - 1
        lane = jax.lax.broadcasted_iota(jnp.int32, (N, 128), 1)
        xm = jnp.where(lane < TAILW, x[:, NFULL * 128:(NFULL + 1) * 128], 0.0)
        sa = xm
        ga = jnp.where(td == jj * F + NFULL, xm, 0.0)
        for f in range(NFULL):
            xs = x[:, f * 128:(f + 1) * 128]
            sa = sa + xs
            ga = ga + jnp.where(td == jj * F + f, xs, 0.0)
        sa = sa_ref[...] + sa
        ga = ga_ref[...] + ga
        s = jnp.sum(sa, axis=1, keepdims=True)
        g = jnp.sum(jnp.where(lane == tm, ga, 0.0), axis=1, keepdims=True)
        term = C0 + U * p0_ref[...] + (U - CONF) * g - U * s
        nonpad = jnp.logical_or(td != 0, tm != 0)
        out_ref[0, 0] = jnp.sum(jnp.where(nonpad, term, 0.0)) / N


def kernel(preds, trgs):
    t = trgs.astype(jnp.int32)
    td = (t // 128).reshape(N, 1)
    tm = (t % 128).reshape(N, 1)
    out = pl.pallas_call(
        _tc_body,
        grid=(NBLK,),
        in_specs=[
            pl.BlockSpec((N, 1), lambda j: (0, 0)),
            pl.BlockSpec((N, 1), lambda j: (0, 0)),
            pl.BlockSpec((N, CB), lambda j: (0, j)),
        ],
        out_specs=pl.BlockSpec((1, 1), lambda j: (0, 0), memory_space=pltpu.SMEM),
        out_shape=jax.ShapeDtypeStruct((1, 1), jnp.float32),
        scratch_shapes=[
            pltpu.VMEM((N, 128), jnp.float32),
            pltpu.VMEM((N, 128), jnp.float32),
            pltpu.VMEM((N, 1), jnp.float32),
        ],
        compiler_params=pltpu.CompilerParams(
            dimension_semantics=("arbitrary",),
        ),
    )(td, tm, preds)
    return out[0, 0]


# speedup vs baseline: 1.9421x; 1.0623x over previous
import functools
import math

import jax
import jax.numpy as jnp
from jax import lax
from jax.experimental import pallas as pl
from jax.experimental.pallas import tpu as pltpu
from jax.experimental.pallas import tpu_sc as plsc

N = 1024
V = 100000
PAD = 0
EPS = 0.1
CONF = 1.0 - EPS
U = EPS / (V - 2)
C0 = EPS * math.log(U) + CONF * math.log(CONF)

CW = 1024
K_SC = 24
C_SC = K_SC * CW
CB = 4096
NT = (V - C_SC + CB - 1) // CB
TCW = V - C_SC


NW = 32
RPW = N // NW
L = 16


def _lane_total(a, lane):
    for sh in (8, 4, 2, 1):
        a = a + jnp.take(a, jnp.bitwise_xor(lane, sh))
    return a


@functools.partial(
    pl.kernel,
    mesh=plsc.VectorSubcoreMesh(core_axis_name="c", subcore_axis_name="s"),
    out_type=jax.ShapeDtypeStruct((N,), jnp.float32),
    scratch_types=[
        pltpu.VMEM((RPW,), jnp.int32),
        pltpu.VMEM((RPW, CW), jnp.float32),
        pltpu.VMEM((RPW, L), jnp.float32),
        pltpu.VMEM((RPW, L), jnp.float32),
        pltpu.VMEM((RPW,), jnp.float32),
    ],
)
def _sc_part(preds_hbm, trg_hbm, out_hbm, trg_v, buf, accv, gv, r_v):
    wid = lax.axis_index("s") * 2 + lax.axis_index("c")
    base = wid * RPW
    pltpu.sync_copy(trg_hbm.at[pl.ds(base, RPW)], trg_v)
    zero = jnp.zeros((L,), jnp.float32)
    lane = lax.iota(jnp.int32, L)
    for r in range(RPW):
        accv[r, :] = zero
        gv[r, :] = zero
    t_half = [trg_v[pl.ds(0, L)], trg_v[pl.ds(L, L)]]

    def chunk_body(c, carry):
        cb = c * CW
        pltpu.sync_copy(preds_hbm.at[pl.ds(base, RPW), pl.ds(cb, CW)], buf)
        colbase = jnp.full((L,), cb, jnp.int32) + lane
        for r in range(RPW):
            tb = jnp.take(t_half[r // L], jnp.full((L,), r % L, jnp.int32))
            d = tb - colbase
            acc = accv[r, :]
            g = gv[r, :]
            for k in range(CW // L):
                v = buf[r, pl.ds(k * L, L)]
                acc = acc + v
                g = g + jnp.where(d == k * L, v, 0.0)
            accv[r, :] = acc
            gv[r, :] = g
        return carry

    lax.fori_loop(0, K_SC, chunk_body, 0)

    pltpu.sync_copy(
        preds_hbm.at[pl.ds(base, RPW), pl.ds(0, 128)], buf.at[:, pl.ds(0, 128)]
    )
    idx0 = jnp.zeros((L,), jnp.int32)
    res = [zero, zero]
    for r in range(RPW):
        a = _lane_total(accv[r, :], lane)
        g = _lane_total(gv[r, :], lane)
        p0 = jnp.take(buf[r, pl.ds(0, L)], idx0)
        val = C0 + U * p0 + (U - CONF) * g - U * a
        h = r // L
        res[h] = jnp.where(lane == r % L, val, res[h])
    for h in range(2):
        r_out = jnp.where(t_half[h] != PAD, res[h], 0.0)
        r_v[pl.ds(h * L, L)] = r_out
    pltpu.sync_copy(r_v, out_hbm.at[pl.ds(base, RPW)])


def _tc_body(trg_ref, x_ref, out_ref, acc_ref, gacc_ref):
    j = pl.program_id(0)
    x = x_ref[...]
    trg = trg_ref[...]
    col = jax.lax.broadcasted_iota(jnp.int32, (N, CB), 1) + C_SC + j * CB
    xv = jnp.where(col < V, x, 0.0)
    acc = jnp.sum(xv, axis=1, keepdims=True)
    g = jnp.sum(jnp.where(col == trg, xv, 0.0), axis=1, keepdims=True)

    @pl.when(j == 0)
    def _init():
        acc_ref[...] = acc
        gacc_ref[...] = g

    @pl.when(j > 0)
    def _accum():
        acc_ref[...] += acc
        gacc_ref[...] += g

    @pl.when(j == NT - 1)
    def _final():
        term = (U - CONF) * gacc_ref[...] - U * acc_ref[...]
        out_ref[0, 0] = jnp.sum(jnp.where(trg != PAD, term, 0.0))


def _combine_body(t1_ref, rsc_ref, out_ref):
    out_ref[0, 0] = (t1_ref[0, 0] + jnp.sum(rsc_ref[...])) / N


def kernel(preds, trgs):
    trgs32 = trgs.astype(jnp.int32)
    r_sc = _sc_part(preds, trgs32)
    t1 = pl.pallas_call(
        _tc_body,
        grid=(NT,),
        in_specs=[
            pl.BlockSpec((N, 1), lambda j: (0, 0)),
            pl.BlockSpec((N, CB), lambda j: (0, j + C_SC // CB)),
        ],
        out_specs=pl.BlockSpec((1, 1), lambda j: (0, 0), memory_space=pltpu.SMEM),
        out_shape=jax.ShapeDtypeStruct((1, 1), jnp.float32),
        scratch_shapes=[
            pltpu.VMEM((N, 1), jnp.float32),
            pltpu.VMEM((N, 1), jnp.float32),
        ],
        compiler_params=pltpu.CompilerParams(
            dimension_semantics=("arbitrary",),
        ),
    )(trgs32.reshape(N, 1), preds)
    out = pl.pallas_call(
        _combine_body,
        in_specs=[
            pl.BlockSpec(memory_space=pltpu.SMEM),
            pl.BlockSpec((8, 128), lambda: (0, 0)),
        ],
        out_specs=pl.BlockSpec(memory_space=pltpu.SMEM),
        out_shape=jax.ShapeDtypeStruct((1, 1), jnp.float32),
    )(t1, r_sc.reshape(8, 128))
    return out[0, 0]
